# Initial kernel scaffold; baseline (speedup 1.0000x reference)
import jax
import jax.numpy as jnp
from jax.experimental import pallas as pl


def kernel(expert_outputs, gates):
    raise NotImplementedError("write your pallas kernel here")


# speedup vs baseline: 41.5891x; 41.5891x over previous
import jax
import jax.numpy as jnp
from jax.experimental import pallas as pl
from jax.experimental.pallas import tpu as pltpu


def _combine_body(g_ref, e_ref, o_ref):
    o_ref[...] = jnp.dot(g_ref[...], e_ref[...],
                         preferred_element_type=jnp.float32)


def kernel(expert_outputs, gates):
    num_images, num_experts = gates.shape
    d = expert_outputs.shape[1]
    bm = 512
    grid = (num_images // bm,)
    return pl.pallas_call(
        _combine_body,
        grid=grid,
        in_specs=[
            pl.BlockSpec((bm, num_experts), lambda i: (i, 0)),
            pl.BlockSpec((num_experts, d), lambda i: (0, 0)),
        ],
        out_specs=pl.BlockSpec((bm, d), lambda i: (i, 0)),
        out_shape=jax.ShapeDtypeStruct((num_images, d), jnp.float32),
    )(gates, expert_outputs)
